# Initial kernel scaffold; baseline (speedup 1.0000x reference)
import jax
import jax.numpy as jnp
from jax import lax
from jax.experimental import pallas as pl
from jax.experimental.pallas import tpu as pltpu
from jax.experimental.pallas import tpu_sc as plsc

KC = 8192
DC = 256
BETA = 0.25

BN = 1024
BK = 1024


def _argmin_body(x_ref, emb_ref, idx_ref, best_ref, bidx_ref):
    kb = pl.program_id(1)

    @pl.when(kb == 0)
    def _init():
        best_ref[...] = jnp.full((BN,), jnp.inf, jnp.float32)
        bidx_ref[...] = jnp.zeros((BN,), jnp.int32)

    x = x_ref[...]
    emb = emb_ref[...]
    dot = lax.dot_general(x, emb, (((1,), (1,)), ((), ())),
                          preferred_element_type=jnp.float32)
    csb = jnp.sum(emb * emb, axis=1)
    isb = jnp.sum(x * x, axis=1)
    dis = (csb[None, :] + isb[:, None]) - 2.0 * dot
    m = jnp.min(dis, axis=1)
    ii = lax.broadcasted_iota(jnp.int32, (BN, BK), 1)
    cand = jnp.where(dis == m[:, None], ii, BK)
    lidx = jnp.min(cand, axis=1) + kb * BK
    better = m < best_ref[...]
    bidx_ref[...] = jnp.where(better, lidx, bidx_ref[...])
    best_ref[...] = jnp.where(better, m, best_ref[...])

    @pl.when(kb == KC // BK - 1)
    def _emit():
        idx_ref[...] = bidx_ref[...].reshape(BN // 128, 128)


def _argmin_call(xt, emb):
    n = xt.shape[0]
    return pl.pallas_call(
        _argmin_body,
        grid=(n // BN, KC // BK),
        in_specs=[
            pl.BlockSpec((BN, DC), lambda i, k: (i, 0)),
            pl.BlockSpec((BK, DC), lambda i, k: (k, 0)),
        ],
        out_specs=pl.BlockSpec((BN // 128, 128), lambda i, k: (i, 0)),
        out_shape=jax.ShapeDtypeStruct((n // 128, 128), jnp.int32),
        scratch_shapes=[
            pltpu.VMEM((BN,), jnp.float32),
            pltpu.VMEM((BN,), jnp.int32),
        ],
        compiler_params=pltpu.CompilerParams(
            dimension_semantics=("arbitrary", "arbitrary"),
        ),
    )(xt, emb)


_NC = 2
_NS = 16
_NW = _NC * _NS
_CHUNK = 128
_TPW = 8192 // _NW
_NJ = _TPW // _CHUNK


def _sc_gather_body(emb_hbm, idx_hbm, quant_hbm, counts_hbm,
                    idx_v, rows_v, ones_v, zero_v, shared, sem):
    c = lax.axis_index("c")
    s = lax.axis_index("s")
    wid = s * _NC + c
    base = wid * _NJ

    pltpu.sync_copy(idx_hbm.at[pl.ds(base, _NJ)], idx_v)

    cps = [pltpu.async_copy(emb_hbm.at[idx_v.at[j]], rows_v.at[j], sem)
           for j in range(_NJ)]

    for u in range(_CHUNK // 16):
        ones_v[pl.ds(u * 16, 16)] = jnp.full((16,), 1.0, jnp.float32)
        zero_v[pl.ds(u * 16, 16)] = jnp.zeros((16,), jnp.float32)

    per_sub = KC // _NS
    for u in range(per_sub // _CHUNK):
        pltpu.sync_copy(zero_v, shared.at[pl.ds(s * per_sub + u * _CHUNK,
                                                _CHUNK)])
    plsc.subcore_barrier()
    for j in range(_NJ):
        pltpu.sync_copy(ones_v, shared.at[idx_v.at[j]], add=True)
    plsc.subcore_barrier()

    @pl.when(s == 0)
    def _emit_counts():
        pltpu.sync_copy(shared, counts_hbm.at[c])

    for cp in cps:
        cp.wait()
    pltpu.sync_copy(rows_v, quant_hbm.at[pl.ds(base, _NJ)])


def _sc_gather_call(emb, idx2d):
    mesh = plsc.VectorSubcoreMesh(core_axis_name="c", subcore_axis_name="s")
    kern = pl.kernel(
        _sc_gather_body,
        out_type=[
            jax.ShapeDtypeStruct((64, _CHUNK, DC), jnp.float32),
            jax.ShapeDtypeStruct((_NC, KC), jnp.float32),
        ],
        mesh=mesh,
        scratch_types=[
            pltpu.VMEM((_NJ, _CHUNK), jnp.int32),
            pltpu.VMEM((_NJ, _CHUNK, DC), jnp.float32),
            pltpu.VMEM((_CHUNK,), jnp.float32),
            pltpu.VMEM((_CHUNK,), jnp.float32),
            pltpu.VMEM_SHARED((KC,), jnp.float32),
            pltpu.SemaphoreType.DMA,
        ],
    )
    return kern(emb, idx2d)


BN3 = 1024


def _finish_body(x_ref, q_ref, cnt_ref, y_ref, loss_ref, perp_ref, acc_ref):
    i = pl.program_id(0)

    @pl.when(i == 0)
    def _init():
        acc_ref[0, 0] = 0.0

    x = x_ref[...]
    q = q_ref[...]
    d = q - x
    y_ref[...] = x + d
    acc_ref[0, 0] += jnp.sum(d * d)

    @pl.when(i == pl.num_programs(0) - 1)
    def _emit():
        n_total = pl.num_programs(0) * BN3 * DC
        loss_ref[0, 0] = BETA * (acc_ref[0, 0] / n_total)
        cnt = cnt_ref[0:1, :] + cnt_ref[1:2, :]
        p = cnt * (1.0 / 8192.0)
        ent = jnp.sum(p * jnp.log(p + 1e-10))
        perp_ref[0, 0] = jnp.exp(-1.0 * ent)


def _finish_call(xt, quant, counts):
    n = xt.shape[0]
    return pl.pallas_call(
        _finish_body,
        grid=(n // BN3,),
        in_specs=[
            pl.BlockSpec((BN3, DC), lambda i: (i, 0)),
            pl.BlockSpec((BN3, DC), lambda i: (i, 0)),
            pl.BlockSpec((_NC, KC), lambda i: (0, 0)),
        ],
        out_specs=[
            pl.BlockSpec((BN3, DC), lambda i: (i, 0)),
            pl.BlockSpec(memory_space=pltpu.SMEM),
            pl.BlockSpec(memory_space=pltpu.SMEM),
        ],
        out_shape=[
            jax.ShapeDtypeStruct((n, DC), jnp.float32),
            jax.ShapeDtypeStruct((1, 1), jnp.float32),
            jax.ShapeDtypeStruct((1, 1), jnp.float32),
        ],
        scratch_shapes=[pltpu.SMEM((1, 1), jnp.float32)],
        compiler_params=pltpu.CompilerParams(
            dimension_semantics=("arbitrary",),
        ),
    )(xt, quant, counts)


def kernel(x, embedding):
    b, d, t = x.shape
    xt = jnp.transpose(x, (0, 2, 1)).reshape(-1, d)
    idx2d = _argmin_call(xt, embedding)
    quant3, counts = _sc_gather_call(embedding, idx2d)
    quant = quant3.reshape(-1, d)
    y, loss, perp = _finish_call(xt, quant, counts)
    yout = jnp.transpose(y.reshape(b, t, d), (0, 2, 1))
    return yout, loss.reshape(()), perp.reshape(())


# speedup vs baseline: 1.1526x; 1.1526x over previous
import jax
import jax.numpy as jnp
from jax import lax
from jax.experimental import pallas as pl
from jax.experimental.pallas import tpu as pltpu
from jax.experimental.pallas import tpu_sc as plsc

KC = 8192
DC = 256
BETA = 0.25

BN = 1024
BK = 512
_L = 128


def _argmin_body(xf_ref, xb_ref, ef_ref, eb_ref, idx_ref,
                 vm_ref, vi_ref, isb_ref):
    kb = pl.program_id(1)
    ones8 = jnp.ones((8, DC), jnp.float32)

    @pl.when(kb == 0)
    def _init():
        vm_ref[...] = jnp.full((BN, _L), jnp.inf, jnp.float32)
        vi_ref[...] = jnp.zeros((BN, _L), jnp.int32)
        x2 = xf_ref[...] * xf_ref[...]
        is8 = lax.dot_general(x2, ones8, (((1,), (1,)), ((), ())),
                              preferred_element_type=jnp.float32)
        isb_ref[...] = jnp.broadcast_to(is8[:, 0:1], (BN, _L))

    e = ef_ref[...]
    e2 = e * e
    cs8 = lax.dot_general(ones8, e2, (((1,), (1,)), ((), ())),
                          preferred_element_type=jnp.float32)
    xb = xb_ref[...]
    isb = isb_ref[...]
    vm = vm_ref[...]
    vi = vi_ref[...]
    lane = lax.broadcasted_iota(jnp.int32, (BN, _L), 1)
    for g in range(BK // _L):
        eg = eb_ref[pl.ds(g * _L, _L), :]
        dg = lax.dot_general(xb, eg, (((1,), (1,)), ((), ())),
                             preferred_element_type=jnp.float32)
        t = cs8[0:1, g * _L:(g + 1) * _L] + isb
        dis = t - 2.0 * dg
        gidx = lane + (kb * BK + g * _L)
        better = dis < vm
        vm = jnp.where(better, dis, vm)
        vi = jnp.where(better, gidx, vi)
    vm_ref[...] = vm
    vi_ref[...] = vi

    @pl.when(kb == KC // BK - 1)
    def _emit():
        v, ix = vm, vi
        for sh in (1, 2, 4, 8, 16, 32, 64):
            rv = pltpu.roll(v, sh, 1)
            ri = pltpu.roll(ix, sh, 1)
            b = (rv < v) | ((rv == v) & (ri < ix))
            v = jnp.where(b, rv, v)
            ix = jnp.where(b, ri, ix)
        idx_ref[...] = ix[:, 0]


def _argmin_call(xt, emb):
    n = xt.shape[0]
    xb16 = xt.astype(jnp.bfloat16)
    eb16 = emb.astype(jnp.bfloat16)
    return pl.pallas_call(
        _argmin_body,
        grid=(n // BN, KC // BK),
        in_specs=[
            pl.BlockSpec((BN, DC), lambda i, k: (i, 0)),
            pl.BlockSpec((BN, DC), lambda i, k: (i, 0)),
            pl.BlockSpec((BK, DC), lambda i, k: (k, 0)),
            pl.BlockSpec((BK, DC), lambda i, k: (k, 0)),
        ],
        out_specs=pl.BlockSpec((BN,), lambda i, k: (i,)),
        out_shape=jax.ShapeDtypeStruct((n,), jnp.int32),
        scratch_shapes=[
            pltpu.VMEM((BN, _L), jnp.float32),
            pltpu.VMEM((BN, _L), jnp.int32),
            pltpu.VMEM((BN, _L), jnp.float32),
        ],
        compiler_params=pltpu.CompilerParams(
            dimension_semantics=("arbitrary", "arbitrary"),
        ),
    )(xt, xb16, emb, eb16)


_NC = 2
_NS = 16
_NW = _NC * _NS
_CHUNK = 128
_TPW = 8192 // _NW
_NJ = _TPW // _CHUNK


def _sc_gather_body(emb_hbm, idx_hbm, quant_hbm, counts_hbm,
                    idx_v, rows_v, ones_v, zero_v, shared, sem):
    c = lax.axis_index("c")
    s = lax.axis_index("s")
    wid = s * _NC + c
    base = wid * _NJ

    pltpu.sync_copy(idx_hbm.at[pl.ds(base, _NJ)], idx_v)

    cps = [pltpu.async_copy(emb_hbm.at[idx_v.at[j]], rows_v.at[j], sem)
           for j in range(_NJ)]

    for u in range(_CHUNK // 16):
        ones_v[pl.ds(u * 16, 16)] = jnp.full((16,), 1.0, jnp.float32)
        zero_v[pl.ds(u * 16, 16)] = jnp.zeros((16,), jnp.float32)

    per_sub = KC // _NS
    for u in range(per_sub // _CHUNK):
        pltpu.sync_copy(zero_v, shared.at[pl.ds(s * per_sub + u * _CHUNK,
                                                _CHUNK)])
    plsc.subcore_barrier()
    for j in range(_NJ):
        pltpu.sync_copy(ones_v, shared.at[idx_v.at[j]], add=True)
    plsc.subcore_barrier()

    @pl.when(s == 0)
    def _emit_counts():
        pltpu.sync_copy(shared, counts_hbm.at[c])

    for cp in cps:
        cp.wait()
    pltpu.sync_copy(rows_v, quant_hbm.at[pl.ds(base, _NJ)])


def _sc_gather_call(emb, idx2d):
    mesh = plsc.VectorSubcoreMesh(core_axis_name="c", subcore_axis_name="s",
                                  num_cores=_NC, num_subcores=_NS)
    kern = pl.kernel(
        _sc_gather_body,
        out_type=[
            jax.ShapeDtypeStruct((64, _CHUNK, DC), jnp.float32),
            jax.ShapeDtypeStruct((_NC, KC), jnp.float32),
        ],
        mesh=mesh,
        scratch_types=[
            pltpu.VMEM((_NJ, _CHUNK), jnp.int32),
            pltpu.VMEM((_NJ, _CHUNK, DC), jnp.float32),
            pltpu.VMEM((_CHUNK,), jnp.float32),
            pltpu.VMEM((_CHUNK,), jnp.float32),
            pltpu.VMEM_SHARED((KC,), jnp.float32),
            pltpu.SemaphoreType.DMA,
        ],
    )
    return kern(emb, idx2d)


BN3 = 1024


def _finish_body(x_ref, q_ref, cnt_ref, y_ref, loss_ref, perp_ref, acc_ref):
    i = pl.program_id(0)

    @pl.when(i == 0)
    def _init():
        acc_ref[0, 0] = 0.0

    x = x_ref[...]
    q = q_ref[...]
    d = q - x
    y_ref[...] = x + d
    acc_ref[0, 0] += jnp.sum(d * d)

    @pl.when(i == pl.num_programs(0) - 1)
    def _emit():
        n_total = pl.num_programs(0) * BN3 * DC
        loss_ref[0, 0] = BETA * (acc_ref[0, 0] / n_total)
        cnt = cnt_ref[0:1, :] + cnt_ref[1:2, :]
        p = cnt * (1.0 / 8192.0)
        ent = jnp.sum(p * jnp.log(p + 1e-10))
        perp_ref[0, 0] = jnp.exp(-1.0 * ent)


def _finish_call(xt, quant, counts):
    n = xt.shape[0]
    return pl.pallas_call(
        _finish_body,
        grid=(n // BN3,),
        in_specs=[
            pl.BlockSpec((BN3, DC), lambda i: (i, 0)),
            pl.BlockSpec((BN3, DC), lambda i: (i, 0)),
            pl.BlockSpec((_NC, KC), lambda i: (0, 0)),
        ],
        out_specs=[
            pl.BlockSpec((BN3, DC), lambda i: (i, 0)),
            pl.BlockSpec(memory_space=pltpu.SMEM),
            pl.BlockSpec(memory_space=pltpu.SMEM),
        ],
        out_shape=[
            jax.ShapeDtypeStruct((n, DC), jnp.float32),
            jax.ShapeDtypeStruct((1, 1), jnp.float32),
            jax.ShapeDtypeStruct((1, 1), jnp.float32),
        ],
        scratch_shapes=[pltpu.SMEM((1, 1), jnp.float32)],
        compiler_params=pltpu.CompilerParams(
            dimension_semantics=("arbitrary",),
        ),
    )(xt, quant, counts)


def kernel(x, embedding):
    b, d, t = x.shape
    xt = jnp.transpose(x, (0, 2, 1)).reshape(-1, d)
    idx2d = _argmin_call(xt, embedding).reshape(-1, 128)
    quant3, counts = _sc_gather_call(embedding, idx2d)
    quant = quant3.reshape(-1, d)
    y, loss, perp = _finish_call(xt, quant, counts)
    yout = jnp.transpose(y.reshape(b, t, d), (0, 2, 1))
    return yout, loss.reshape(()), perp.reshape(())
